# Initial kernel scaffold; baseline (speedup 1.0000x reference)
import jax
import jax.numpy as jnp
from jax.experimental import pallas as pl


def kernel(x, emb_table):
    raise NotImplementedError("write your pallas kernel here")


# speedup vs baseline: 1.5172x; 1.5172x over previous
import functools

import jax
import jax.numpy as jnp
from jax import lax
from jax.experimental import pallas as pl
from jax.experimental.pallas import tpu as pltpu
from jax.experimental.pallas import tpu_sc as plsc

NC = 2
NS = 16
NW = NC * NS
LANES = 16

BATCH = 16384
CTX = 20
EMB = 32

B_PER_W = BATCH // NW
ROWS_PER_CHUNK = 4
IDX_PER_CHUNK = ROWS_PER_CHUNK * CTX
N_CHUNKS = B_PER_W // ROWS_PER_CHUNK


def _cbow_body(table_hbm, x_hbm, out_hbm, idx_v, rows_v, out_v, sem):
    wid = lax.axis_index("s") * NC + lax.axis_index("c")

    pltpu.sync_copy(x_hbm.at[wid], idx_v)

    inv_ctx = jnp.float32(1.0 / CTX)

    def chunk(j, carry):
        pltpu.async_copy(table_hbm.at[idx_v.at[j]], rows_v, sem).wait()
        for r in range(ROWS_PER_CHUNK):
            base = r * CTX
            lo = rows_v[base, pl.ds(0, LANES)]
            hi = rows_v[base, pl.ds(LANES, LANES)]
            for c in range(1, CTX):
                lo = lo + rows_v[base + c, pl.ds(0, LANES)]
                hi = hi + rows_v[base + c, pl.ds(LANES, LANES)]
            row = j * ROWS_PER_CHUNK + r
            out_v[row, pl.ds(0, LANES)] = lo * inv_ctx
            out_v[row, pl.ds(LANES, LANES)] = hi * inv_ctx
        return carry

    lax.fori_loop(0, N_CHUNKS, chunk, 0)

    pltpu.sync_copy(out_v, out_hbm.at[wid])


@jax.jit
def _cbow(x3, emb_table):
    mesh = plsc.VectorSubcoreMesh(
        core_axis_name="c", subcore_axis_name="s",
        num_cores=NC, num_subcores=NS)
    f = functools.partial(
        pl.kernel,
        out_type=jax.ShapeDtypeStruct((NW, B_PER_W, EMB), jnp.float32),
        mesh=mesh,
        scratch_types=[
            pltpu.VMEM((N_CHUNKS, IDX_PER_CHUNK), jnp.int32),
            pltpu.VMEM((IDX_PER_CHUNK, EMB), jnp.float32),
            pltpu.VMEM((B_PER_W, EMB), jnp.float32),
            pltpu.SemaphoreType.DMA,
        ],
        compiler_params=pltpu.CompilerParams(use_tc_tiling_on_sc=False),
    )(_cbow_body)
    return f(emb_table, x3)


def kernel(x, emb_table):
    x3 = x.astype(jnp.int32).reshape(NW, N_CHUNKS, IDX_PER_CHUNK)
    out = _cbow(x3, emb_table)
    return out.reshape(BATCH, EMB)


# speedup vs baseline: 1.7199x; 1.1336x over previous
import functools

import jax
import jax.numpy as jnp
from jax import lax
from jax.experimental import pallas as pl
from jax.experimental.pallas import tpu as pltpu
from jax.experimental.pallas import tpu_sc as plsc

NC = 2
NS = 16
NW = NC * NS
LANES = 16

BATCH = 16384
CTX = 20
EMB = 32

B_PER_W = BATCH // NW
ROWS_PER_CHUNK = 4
IDX_PER_CHUNK = ROWS_PER_CHUNK * CTX
N_CHUNKS = B_PER_W // ROWS_PER_CHUNK


NBUF = 4
N_STEPS = N_CHUNKS // NBUF


def _cbow_body(table_hbm, x_hbm, out_hbm, idx_v, rows_v, out_v,
               sem0, sem1, sem2, sem3):
    wid = lax.axis_index("s") * NC + lax.axis_index("c")
    sems = (sem0, sem1, sem2, sem3)

    pltpu.sync_copy(x_hbm.at[wid], idx_v)

    inv_ctx = jnp.float32(1.0 / CTX)

    for b in range(NBUF):
        pltpu.async_copy(table_hbm.at[idx_v.at[b]], rows_v.at[b], sems[b])

    def step(g, carry):
        for b in range(NBUF):
            j = g * NBUF + b
            pltpu.make_async_copy(
                table_hbm.at[idx_v.at[j]], rows_v.at[b], sems[b]).wait()
            for r in range(ROWS_PER_CHUNK):
                base = r * CTX
                lo = rows_v[b, base, pl.ds(0, LANES)]
                hi = rows_v[b, base, pl.ds(LANES, LANES)]
                for c in range(1, CTX):
                    lo = lo + rows_v[b, base + c, pl.ds(0, LANES)]
                    hi = hi + rows_v[b, base + c, pl.ds(LANES, LANES)]
                row = j * ROWS_PER_CHUNK + r
                out_v[row, pl.ds(0, LANES)] = lo * inv_ctx
                out_v[row, pl.ds(LANES, LANES)] = hi * inv_ctx

            @pl.when(g < N_STEPS - 1)
            def _():
                pltpu.async_copy(
                    table_hbm.at[idx_v.at[j + NBUF]], rows_v.at[b], sems[b])
        return carry

    lax.fori_loop(0, N_STEPS, step, 0)

    pltpu.sync_copy(out_v, out_hbm.at[wid])


@jax.jit
def _cbow(x3, emb_table):
    mesh = plsc.VectorSubcoreMesh(
        core_axis_name="c", subcore_axis_name="s",
        num_cores=NC, num_subcores=NS)
    f = functools.partial(
        pl.kernel,
        out_type=jax.ShapeDtypeStruct((NW, B_PER_W, EMB), jnp.float32),
        mesh=mesh,
        scratch_types=[
            pltpu.VMEM((N_CHUNKS, IDX_PER_CHUNK), jnp.int32),
            pltpu.VMEM((NBUF, IDX_PER_CHUNK, EMB), jnp.float32),
            pltpu.VMEM((B_PER_W, EMB), jnp.float32),
            pltpu.SemaphoreType.DMA,
            pltpu.SemaphoreType.DMA,
            pltpu.SemaphoreType.DMA,
            pltpu.SemaphoreType.DMA,
        ],
        compiler_params=pltpu.CompilerParams(use_tc_tiling_on_sc=False),
    )(_cbow_body)
    return f(emb_table, x3)


def kernel(x, emb_table):
    x3 = x.astype(jnp.int32).reshape(NW, N_CHUNKS, IDX_PER_CHUNK)
    out = _cbow(x3, emb_table)
    return out.reshape(BATCH, EMB)
